# Initial kernel scaffold; baseline (speedup 1.0000x reference)
import jax
import jax.numpy as jnp
from jax.experimental import pallas as pl


def kernel(x, edge_index, W1, b1, g1, be1, W2, b2, g2, be2, W3, b3, gbn, bbn, Wm, bm, Wv, bv):
    raise NotImplementedError("write your pallas kernel here")


# speedup vs baseline: 8.0050x; 8.0050x over previous
import functools

import jax
import jax.numpy as jnp
from jax import lax
from jax.experimental import pallas as pl
from jax.experimental.pallas import tpu as pltpu
from jax.experimental.pallas import tpu_sc as plsc

NC = 2
NS = 16
NW = NC * NS
CH = 128


def _sc_edge_aggregate(x, src_p, dst_p, n_acc, k):
    D = x.shape[1]
    rpt = n_acc // NS
    mesh = plsc.VectorSubcoreMesh(core_axis_name="c", subcore_axis_name="s")

    @functools.partial(
        pl.kernel,
        out_type=(
            jax.ShapeDtypeStruct((n_acc, D), jnp.float32),
            jax.ShapeDtypeStruct((n_acc, D), jnp.float32),
        ),
        mesh=mesh,
        scratch_types=[
            pltpu.VMEM((k, CH), jnp.int32),
            pltpu.VMEM((k, CH), jnp.int32),
            pltpu.VMEM((CH, D), jnp.float32),
            pltpu.VMEM_SHARED((n_acc, D), jnp.float32),
            pltpu.SemaphoreType.DMA,
        ],
    )
    def agg(x_hbm, src_hbm, dst_hbm, out0_hbm, out1_hbm,
            src_v, dst_v, rows_v, acc_sh, sem):
        cid = lax.axis_index("c")
        sid = lax.axis_index("s")
        wid = sid * NC + cid

        zvec = jnp.zeros((16,), jnp.float32)

        def zrow(i, carry):
            for l in range(D // 16):
                rows_v[i, pl.ds(l * 16, 16)] = zvec
            return carry

        lax.fori_loop(0, CH, zrow, 0)
        for r in range(rpt // CH):
            pltpu.sync_copy(rows_v, acc_sh.at[pl.ds(sid * rpt + r * CH, CH)])
        plsc.subcore_barrier()

        pltpu.sync_copy(src_hbm.at[wid], src_v)
        pltpu.sync_copy(dst_hbm.at[wid], dst_v)

        def body(j, carry):
            pltpu.async_copy(x_hbm.at[src_v.at[j]], rows_v, sem).wait()
            pltpu.sync_copy(rows_v, acc_sh.at[dst_v.at[j]], add=True)
            return carry

        lax.fori_loop(0, k, body, 0)
        plsc.subcore_barrier()

        @pl.when(cid == 0)
        def _():
            pltpu.sync_copy(acc_sh.at[pl.ds(sid * rpt, rpt)],
                            out0_hbm.at[pl.ds(sid * rpt, rpt)])

        @pl.when(cid == 1)
        def _():
            pltpu.sync_copy(acc_sh.at[pl.ds(sid * rpt, rpt)],
                            out1_hbm.at[pl.ds(sid * rpt, rpt)])

    return agg(x, src_p, dst_p)


def _tc_mlp(x, p0, p1, W1, b1, W2, b2, W3, b3, Wm, bm, Wv, bv, block_rows):
    n, d = x.shape
    h_dim = W1.shape[1]
    grid = (pl.cdiv(n, block_rows),)

    def mm(h, w):
        return lax.dot_general(h, w, (((1,), (0,)), ((), ())),
                               preferred_element_type=jnp.float32,
                               precision=lax.Precision.HIGHEST)

    def body(x_r, p0_r, p1_r, W1_r, b1_r, W2_r, b2_r, W3_r, b3_r,
             Wm_r, bm_r, Wv_r, bv_r, mean_r, var_r):
        h = x_r[...] + p0_r[...] + p1_r[...]
        h = jnp.maximum(mm(h, W1_r[...]) + b1_r[...], 0.0)
        h = jnp.maximum(mm(h, W2_r[...]) + b2_r[...], 0.0)
        h = jnp.maximum(mm(h, W3_r[...]) + b3_r[...], 0.0)
        mean_r[...] = mm(h, Wm_r[...]) + bm_r[...]
        var_r[...] = mm(h, Wv_r[...]) + bv_r[...]

    row_spec = pl.BlockSpec((block_rows, d), lambda i: (i, 0))
    w_spec = pl.BlockSpec((d, h_dim), lambda i: (0, 0))
    b_spec = pl.BlockSpec((h_dim,), lambda i: (0,))
    return pl.pallas_call(
        body,
        grid=grid,
        in_specs=[row_spec, row_spec, row_spec,
                  w_spec, b_spec, w_spec, b_spec, w_spec, b_spec,
                  w_spec, b_spec, w_spec, b_spec],
        out_specs=(pl.BlockSpec((block_rows, h_dim), lambda i: (i, 0)),
                   pl.BlockSpec((block_rows, h_dim), lambda i: (i, 0))),
        out_shape=(jax.ShapeDtypeStruct((n, h_dim), jnp.float32),
                   jax.ShapeDtypeStruct((n, h_dim), jnp.float32)),
    )(x, p0, p1, W1, b1, W2, b2, W3, b3, Wm, bm, Wv, bv)


def kernel(x, edge_index, W1, b1, g1, be1, W2, b2, g2, be2, W3, b3,
           gbn, bbn, Wm, bm, Wv, bv):
    n, d = x.shape
    e = edge_index.shape[1]
    bn_eps = 1e-3

    s1 = g1 / jnp.sqrt(1.0 + bn_eps)
    W1f = W1 * s1[None, :]
    b1f = b1 * s1 + be1
    s2 = g2 / jnp.sqrt(1.0 + bn_eps)
    W2f = W2 * s2[None, :]
    b2f = b2 * s2 + be2
    sbn = gbn / jnp.sqrt(1.0 + bn_eps)
    Wmf = Wm * sbn[:, None]
    bmf = bbn @ Wm + bm
    Wvf = Wv * sbn[:, None]
    bvf = bbn @ Wv + bv

    epw = e // NW
    k = pl.cdiv(epw, CH)
    if k % 2:
        k += 1
    pad = k * CH - epw
    n_acc = n + (-n) % (NS * CH)
    n_dummy = n_acc - n
    src_w = edge_index[0].reshape(NW, epw)
    dst_w = edge_index[1].reshape(NW, epw)
    wids = jnp.arange(NW, dtype=jnp.int32)[:, None]
    lane = jnp.arange(pad, dtype=jnp.int32)[None, :]
    pad_src = (wids * pad + lane) % n
    pad_dst = n + (wids * 7 + lane) % n_dummy
    src_p = jnp.concatenate([src_w, pad_src], axis=1).reshape(NW, k, CH)
    dst_p = jnp.concatenate([dst_w, pad_dst], axis=1).reshape(NW, k, CH)

    p0, p1 = _sc_edge_aggregate(x, src_p, dst_p, n_acc, k)
    return _tc_mlp(x, p0, p1, W1f, b1f, W2f, b2f, W3, b3,
                   Wmf, bmf, Wvf, bvf, block_rows=2048)


# speedup vs baseline: 9.2488x; 1.1554x over previous
import functools

import jax
import jax.numpy as jnp
from jax import lax
from jax.experimental import pallas as pl
from jax.experimental.pallas import tpu as pltpu
from jax.experimental.pallas import tpu_sc as plsc

NC = 2
NS = 16
NW = NC * NS
CH = 128


def _sc_edge_aggregate(x, src_p, dst_p, n_acc, k):
    D = x.shape[1]
    rpt = n_acc // NS
    kp = k // 2
    mesh = plsc.VectorSubcoreMesh(core_axis_name="c", subcore_axis_name="s")

    @functools.partial(
        pl.kernel,
        out_type=(
            jax.ShapeDtypeStruct((n_acc, D), jnp.float32),
            jax.ShapeDtypeStruct((n_acc, D), jnp.float32),
        ),
        mesh=mesh,
        scratch_types=[
            pltpu.VMEM((kp, CH), jnp.int32),
            pltpu.VMEM((kp, CH), jnp.int32),
            pltpu.VMEM((CH, D), jnp.float32),
            pltpu.VMEM((CH, D), jnp.float32),
            pltpu.SemaphoreType.DMA,
            pltpu.SemaphoreType.DMA,
            pltpu.SemaphoreType.DMA,
            pltpu.SemaphoreType.DMA,
            pltpu.SemaphoreType.DMA,
            pltpu.VMEM_SHARED((n_acc, D), jnp.float32),
        ],
    )
    def agg(x_hbm, src_hbm, dst_hbm, out0_hbm, out1_hbm,
            src_v, dst_v, rows0, rows1, isem, gsem0, gsem1, ssem0, ssem1,
            acc_sh):
        cid = lax.axis_index("c")
        sid = lax.axis_index("s")
        wid = sid * NC + cid

        pltpu.async_copy(src_hbm.at[wid, pl.ds(0, kp)], src_v, isem)
        pltpu.async_copy(dst_hbm.at[wid, pl.ds(0, kp)], dst_v, isem)

        zvec = jnp.zeros((16,), jnp.float32)

        def zrow(i, carry):
            for l in range(D // 16):
                rows0[i, pl.ds(l * 16, 16)] = zvec
            return carry

        lax.fori_loop(0, CH, zrow, 0)
        for r in range(rpt // CH):
            pltpu.sync_copy(rows0, acc_sh.at[pl.ds(sid * rpt + r * CH, CH)])
        pltpu.make_async_copy(src_hbm.at[wid, pl.ds(0, kp)], src_v, isem).wait()
        pltpu.make_async_copy(dst_hbm.at[wid, pl.ds(0, kp)], dst_v, isem).wait()
        plsc.subcore_barrier()

        def gather(j, buf, sem):
            pltpu.async_copy(x_hbm.at[src_v.at[j]], buf, sem)

        def gather_wait(j, buf, sem):
            pltpu.make_async_copy(x_hbm.at[src_v.at[j]], buf, sem).wait()

        def scatter(j, buf, sem):
            pltpu.async_copy(buf, acc_sh.at[dst_v.at[j]], sem, add=True)

        def scatter_wait(j, buf, sem):
            pltpu.make_async_copy(buf, acc_sh.at[dst_v.at[j]], sem).wait()

        def body(jj, carry):
            a = 2 * jj
            b = a + 1
            gather_wait(a, rows0, gsem0)
            scatter(a, rows0, ssem0)
            gather_wait(b, rows1, gsem1)
            scatter(b, rows1, ssem1)

            @pl.when(jj < kp // 2 - 1)
            def _():
                scatter_wait(a, rows0, ssem0)
                gather(a + 2, rows0, gsem0)
                scatter_wait(b, rows1, ssem1)
                gather(b + 2, rows1, gsem1)

            return carry

        for phase in range(2):
            if phase:
                pltpu.sync_copy(src_hbm.at[wid, pl.ds(kp, kp)], src_v)
                pltpu.sync_copy(dst_hbm.at[wid, pl.ds(kp, kp)], dst_v)
            gather(0, rows0, gsem0)
            gather(1, rows1, gsem1)
            lax.fori_loop(0, kp // 2, body, 0)
            scatter_wait(kp - 2, rows0, ssem0)
            scatter_wait(kp - 1, rows1, ssem1)
        plsc.subcore_barrier()

        @pl.when(cid == 0)
        def _():
            pltpu.sync_copy(acc_sh.at[pl.ds(sid * rpt, rpt)],
                            out0_hbm.at[pl.ds(sid * rpt, rpt)])

        @pl.when(cid == 1)
        def _():
            pltpu.sync_copy(acc_sh.at[pl.ds(sid * rpt, rpt)],
                            out1_hbm.at[pl.ds(sid * rpt, rpt)])

    return agg(x, src_p, dst_p)


def _tc_mlp(x, p0, p1, W1, b1, W2, b2, W3, b3, Wm, bm, Wv, bv, block_rows):
    n, d = x.shape
    h_dim = W1.shape[1]
    grid = (pl.cdiv(n, block_rows),)

    def mm(h, w):
        return lax.dot_general(h, w, (((1,), (0,)), ((), ())),
                               preferred_element_type=jnp.float32,
                               precision=lax.Precision.HIGHEST)

    def body(x_r, p0_r, p1_r, W1_r, b1_r, W2_r, b2_r, W3_r, b3_r,
             Wm_r, bm_r, Wv_r, bv_r, mean_r, var_r):
        h = x_r[...] + p0_r[...] + p1_r[...]
        h = jnp.maximum(mm(h, W1_r[...]) + b1_r[...], 0.0)
        h = jnp.maximum(mm(h, W2_r[...]) + b2_r[...], 0.0)
        h = jnp.maximum(mm(h, W3_r[...]) + b3_r[...], 0.0)
        mean_r[...] = mm(h, Wm_r[...]) + bm_r[...]
        var_r[...] = mm(h, Wv_r[...]) + bv_r[...]

    row_spec = pl.BlockSpec((block_rows, d), lambda i: (i, 0))
    w_spec = pl.BlockSpec((d, h_dim), lambda i: (0, 0))
    b_spec = pl.BlockSpec((h_dim,), lambda i: (0,))
    return pl.pallas_call(
        body,
        grid=grid,
        in_specs=[row_spec, row_spec, row_spec,
                  w_spec, b_spec, w_spec, b_spec, w_spec, b_spec,
                  w_spec, b_spec, w_spec, b_spec],
        out_specs=(pl.BlockSpec((block_rows, h_dim), lambda i: (i, 0)),
                   pl.BlockSpec((block_rows, h_dim), lambda i: (i, 0))),
        out_shape=(jax.ShapeDtypeStruct((n, h_dim), jnp.float32),
                   jax.ShapeDtypeStruct((n, h_dim), jnp.float32)),
    )(x, p0, p1, W1, b1, W2, b2, W3, b3, Wm, bm, Wv, bv)


def kernel(x, edge_index, W1, b1, g1, be1, W2, b2, g2, be2, W3, b3,
           gbn, bbn, Wm, bm, Wv, bv):
    n, d = x.shape
    e = edge_index.shape[1]
    bn_eps = 1e-3

    s1 = g1 / jnp.sqrt(1.0 + bn_eps)
    W1f = W1 * s1[None, :]
    b1f = b1 * s1 + be1
    s2 = g2 / jnp.sqrt(1.0 + bn_eps)
    W2f = W2 * s2[None, :]
    b2f = b2 * s2 + be2
    sbn = gbn / jnp.sqrt(1.0 + bn_eps)
    Wmf = Wm * sbn[:, None]
    bmf = bbn @ Wm + bm
    Wvf = Wv * sbn[:, None]
    bvf = bbn @ Wv + bv

    epw = e // NW
    k = pl.cdiv(epw, CH)
    k += (-k) % 4
    pad = k * CH - epw
    n_acc = n + (-n) % (NS * CH)
    n_dummy = n_acc - n
    src_w = edge_index[0].reshape(NW, epw)
    dst_w = edge_index[1].reshape(NW, epw)
    wids = jnp.arange(NW, dtype=jnp.int32)[:, None]
    lane = jnp.arange(pad, dtype=jnp.int32)[None, :]
    pad_src = (wids * pad + lane) % n
    pad_dst = n + (wids * 7 + lane) % n_dummy
    src_p = jnp.concatenate([src_w, pad_src], axis=1).reshape(NW, k, CH)
    dst_p = jnp.concatenate([dst_w, pad_dst], axis=1).reshape(NW, k, CH)

    p0, p1 = _sc_edge_aggregate(x, src_p, dst_p, n_acc, k)
    return _tc_mlp(x, p0, p1, W1f, b1f, W2f, b2f, W3, b3,
                   Wmf, bmf, Wvf, bvf, block_rows=2048)
